# baseline (speedup 1.0000x reference)
import functools

import jax
import jax.numpy as jnp
from jax import lax
from jax.experimental import pallas as pl
from jax.experimental.pallas import tpu as pltpu
from jax.experimental.pallas import tpu_sc as plsc

V1 = 1000001
H = 64
B = 16384
L = 200


_ROW_BLK = 8192


def _transform_body(t_ref, w_ref, o_ref):
    o_ref[...] = lax.dot_general(
        t_ref[...], w_ref[...],
        dimension_numbers=(((1,), (1,)), ((), ())),
        preferred_element_type=jnp.float32,
    )


def _transform_table(table, W):
    grid = (pl.cdiv(V1, _ROW_BLK),)
    return pl.pallas_call(
        _transform_body,
        grid=grid,
        in_specs=[
            pl.BlockSpec((_ROW_BLK, H), lambda i: (i, 0)),
            pl.BlockSpec((H, H), lambda i: (0, 0)),
        ],
        out_specs=pl.BlockSpec((_ROW_BLK, H), lambda i: (i, 0)),
        out_shape=jax.ShapeDtypeStruct((V1, H), jnp.float32),
    )(table, W)


_G = 4
_CHUNK = 100
_NCH = (_G * L) // _CHUNK
_NW = 32
_GROUPS = B // _G
_GPW = _GROUPS // _NW


def _sc_mesh():
    return plsc.VectorSubcoreMesh(core_axis_name="c", subcore_axis_name="s")


@functools.partial(
    pl.kernel,
    out_type=jax.ShapeDtypeStruct((B, H), jnp.float32),
    mesh=_sc_mesh(),
    compiler_params=pltpu.CompilerParams(use_tc_tiling_on_sc=False),
    scratch_types=[
        pltpu.VMEM((_NCH, _CHUNK), jnp.int32),
        pltpu.VMEM((_G * L, H), jnp.float32),
        pltpu.VMEM((_G, H), jnp.float32),
        pltpu.VMEM((H,), jnp.float32),
        pltpu.SemaphoreType.DMA,
    ],
)
def _sc_gather_max(x_hbm, table2_hbm, bias_hbm, out_hbm,
                   idx_v, rows_v, out_v, bias_v, sem):
    nc = 2
    wid = lax.axis_index("s") * nc + lax.axis_index("c")

    pltpu.sync_copy(bias_hbm, bias_v)

    def group_body(gi, _):
        g = wid * _GPW + gi
        pltpu.sync_copy(x_hbm.at[g], idx_v)
        copies = []
        for c in range(_NCH):
            copies.append(pltpu.async_copy(
                table2_hbm.at[idx_v.at[c]],
                rows_v.at[pl.ds(c * _CHUNK, _CHUNK)],
                sem,
            ))
        for cp in copies:
            cp.wait()
        for s in range(_G):
            base = s * L

            def red_body(j, acc):
                return tuple(
                    jnp.maximum(acc[h],
                                rows_v[base + j, pl.ds(h * 16, 16)])
                    for h in range(4)
                )

            init = tuple(
                jnp.full((16,), -jnp.inf, jnp.float32) for _ in range(4)
            )
            acc = lax.fori_loop(0, L, red_body, init)
            for h in range(4):
                out_v[s, pl.ds(h * 16, 16)] = (
                    acc[h] + bias_v[pl.ds(h * 16, 16)]
                )
        pltpu.sync_copy(out_v, out_hbm.at[pl.ds(g * _G, _G)])
        return ()

    lax.fori_loop(0, _GPW, group_body, ())


def kernel(x, table, W, b):
    table2 = _transform_table(table, W)
    x3 = x.astype(jnp.int32).reshape(_GROUPS, _NCH, _CHUNK)
    return _sc_gather_max(x3, table2, b)


# speedup vs baseline: 2.8115x; 2.8115x over previous
import functools

import jax
import jax.numpy as jnp
from jax import lax
from jax.experimental import pallas as pl
from jax.experimental.pallas import tpu as pltpu
from jax.experimental.pallas import tpu_sc as plsc

V1 = 1000001
H = 64
B = 16384
L = 200


_CBLK = 8192
_NBLK = (V1 + _CBLK - 1) // _CBLK
_ROWS_PAD = _NBLK * _CBLK


def _transform_body(t_ref, w2_ref, o_ref):
    o_ref[...] = lax.dot_general(
        t_ref[...], w2_ref[...],
        dimension_numbers=(((0,), (0,)), ((), ())),
        preferred_element_type=jnp.float32,
    )


def _transform_table(table, W):
    table_t = table.T
    w2 = jnp.concatenate([W.T, W.T], axis=1)
    out = pl.pallas_call(
        _transform_body,
        grid=(_NBLK,),
        in_specs=[
            pl.BlockSpec((H, _CBLK), lambda i: (0, i)),
            pl.BlockSpec((H, 2 * H), lambda i: (0, 0)),
        ],
        out_specs=pl.BlockSpec((_CBLK, 2 * H), lambda i: (i, 0)),
        out_shape=jax.ShapeDtypeStruct((_ROWS_PAD, 2 * H), jnp.float32),
    )(table_t, w2)
    return out.reshape(2 * _ROWS_PAD, H)


_G = 4
_CHUNK = 100
_NCH = (_G * L) // _CHUNK
_NW = 32
_GROUPS = B // _G
_GPW = _GROUPS // _NW
_OUTBUF_GROUPS = 8


def _sc_mesh():
    return plsc.VectorSubcoreMesh(core_axis_name="c", subcore_axis_name="s")


@functools.partial(
    pl.kernel,
    out_type=jax.ShapeDtypeStruct((B, H), jnp.float32),
    mesh=_sc_mesh(),
    compiler_params=pltpu.CompilerParams(use_tc_tiling_on_sc=False),
    scratch_types=[
        pltpu.VMEM((2, _NCH, _CHUNK), jnp.int32),
        pltpu.VMEM((2, _G * L, H), jnp.float32),
        pltpu.VMEM((_OUTBUF_GROUPS * _G, H), jnp.float32),
        pltpu.VMEM((H,), jnp.float32),
        pltpu.SemaphoreType.DMA,
        pltpu.SemaphoreType.DMA,
        pltpu.SemaphoreType.DMA,
        pltpu.SemaphoreType.DMA,
    ],
)
def _sc_gather_max(x_hbm, table2_hbm, bias_hbm, out_hbm,
                   idx_v, rows_v, out_v, bias_v,
                   isem0, isem1, gsem0, gsem1):
    nc = 2
    wid = lax.axis_index("s") * nc + lax.axis_index("c")
    g_base = wid * _GPW
    g_last = g_base + _GPW - 1
    isems = (isem0, isem1)
    gsems = (gsem0, gsem1)

    pltpu.sync_copy(bias_hbm, bias_v)
    bias = tuple(bias_v[pl.ds(h * 16, 16)] for h in range(4))

    def fire_idx(bb, g):
        pltpu.async_copy(x_hbm.at[g], idx_v.at[bb], isems[bb])

    def wait_idx(bb):
        pltpu.make_async_copy(x_hbm.at[g_base], idx_v.at[bb], isems[bb]).wait()

    def fire_gather(bb):
        wait_idx(bb)
        for c in range(_NCH):
            pltpu.async_copy(
                table2_hbm.at[idx_v.at[bb, c]],
                rows_v.at[bb, pl.ds(c * _CHUNK, _CHUNK)],
                gsems[bb],
            )

    def drain_gather(bb):
        for c in range(_NCH):
            pltpu.make_async_copy(
                table2_hbm.at[idx_v.at[bb, c]],
                rows_v.at[bb, pl.ds(c * _CHUNK, _CHUNK)],
                gsems[bb],
            ).wait()

    def reduce_store(bb, slot):
        def red_body(j, acc):
            new = []
            for s in range(_G):
                row = s * L + j
                for h in range(4):
                    new.append(jnp.maximum(
                        acc[s * 4 + h], rows_v[bb, row, pl.ds(h * 16, 16)]))
            return tuple(new)

        init = tuple(
            jnp.full((16,), -jnp.inf, jnp.float32) for _ in range(_G * 4))
        acc = lax.fori_loop(0, L, red_body, init)
        for s in range(_G):
            for h in range(4):
                out_v[slot * _G + s, pl.ds(h * 16, 16)] = (
                    acc[s * 4 + h] + bias[h])

    fire_idx(0, g_base)
    fire_idx(1, g_base + 1)
    fire_gather(0)

    def body(i2, _):
        ga = g_base + 2 * i2
        gb = ga + 1
        gc = jnp.minimum(ga + 2, g_last)
        gd = jnp.minimum(ga + 3, g_last)
        m = lax.rem(i2, 4)

        fire_gather(1)
        drain_gather(0)
        fire_idx(0, gc)
        reduce_store(0, m * 2)
        fire_gather(0)
        drain_gather(1)
        fire_idx(1, gd)
        reduce_store(1, m * 2 + 1)

        @pl.when(m == 3)
        def _flush():
            row0 = (ga - 6) * _G
            pltpu.sync_copy(
                out_v, out_hbm.at[pl.ds(row0, _OUTBUF_GROUPS * _G)])

        return ()

    lax.fori_loop(0, _GPW // 2, body, ())
    drain_gather(0)
    wait_idx(1)


def kernel(x, table, W, b):
    table2 = _transform_table(table, W)
    x3 = (x.astype(jnp.int32) * 2).reshape(_GROUPS, _NCH, _CHUNK)
    return _sc_gather_max(x3, table2, b)
